# baseline (speedup 1.0000x reference)
import functools

import jax
import jax.numpy as jnp
from jax import lax
from jax.experimental import pallas as pl
from jax.experimental.pallas import tpu as pltpu
from jax.experimental.pallas import tpu_sc as plsc

_K = 16
_FMAX = 3.0e38
_CT = 512


def _knn_kernel(pos_ref, posT_ref, batch_ref, batchT_ref, idx_ref):
    i = pl.program_id(0)
    blk = pos_ref.shape[0]
    n = posT_ref.shape[1]

    pos_b = pos_ref[...]
    px = pos_b[:, 0:1]
    py = pos_b[:, 1:2]
    pz = pos_b[:, 2:3]
    sq_b = px * px + py * py + pz * pz
    batch_b = batch_ref[...]
    rowid = i * blk + lax.broadcasted_iota(jnp.int32, (blk, 1), 0)

    batch_row = batchT_ref[...]
    col_row = lax.broadcasted_iota(jnp.int32, (1, n), 1)
    bmin = jnp.min(batch_b)
    bmax = jnp.max(batch_b)
    start = jnp.min(jnp.where(batch_row == bmin, col_row, n))
    end = jnp.max(jnp.where(batch_row == bmax, col_row + 1, 0))
    t0 = start // _CT
    t1 = (end + _CT - 1) // _CT

    lane16 = lax.broadcasted_iota(jnp.int32, (blk, _K), 1)

    def body(j, carry):
        cur_val, cur_idx = carry
        c = j * _CT
        q = posT_ref[:, pl.ds(c, _CT)]
        qx = q[0:1, :]
        qy = q[1:2, :]
        qz = q[2:3, :]
        dot = px * qx + py * qy + pz * qz
        sq_t = qx * qx + qy * qy + qz * qz
        d = sq_b + sq_t - 2.0 * dot
        gidx = c + lax.broadcasted_iota(jnp.int32, (blk, _CT), 1)
        bt = batchT_ref[:, pl.ds(c, _CT)]
        invalid = (batch_b != bt) | (gidx == rowid)
        d = jnp.where(invalid, _FMAX, d)

        arr_val = jnp.concatenate([cur_val, d], axis=1)
        arr_idx = jnp.concatenate([cur_idx, gidx], axis=1)
        new_val = cur_val
        new_idx = cur_idx
        for r in range(_K):
            mv = jnp.min(arr_val, axis=1, keepdims=True)
            ci = jnp.min(jnp.where(arr_val == mv, arr_idx, jnp.int32(2**30)),
                         axis=1, keepdims=True)
            new_val = jnp.where(lane16 == r, mv, new_val)
            new_idx = jnp.where(lane16 == r, ci, new_idx)
            arr_val = jnp.where((arr_val == mv) & (arr_idx == ci),
                                float("inf"), arr_val)
        return new_val, new_idx

    cur_val = jnp.full((blk, _K), float("inf"), jnp.float32)
    cur_idx = jnp.full((blk, _K), 2**30, jnp.int32)
    _, cur_idx = lax.fori_loop(t0, t1, body, (cur_val, cur_idx))
    idx_ref[...] = cur_idx


def _run_knn(pos, batch_i32, blk):
    n = pos.shape[0]
    grid = n // blk
    return pl.pallas_call(
        _knn_kernel,
        grid=(grid,),
        in_specs=[
            pl.BlockSpec((blk, 3), lambda i: (i, 0)),
            pl.BlockSpec((3, n), lambda i: (0, 0)),
            pl.BlockSpec((blk, 1), lambda i: (i, 0)),
            pl.BlockSpec((1, n), lambda i: (0, 0)),
        ],
        out_specs=pl.BlockSpec((blk, _K), lambda i: (i, 0)),
        out_shape=jax.ShapeDtypeStruct((n, _K), jnp.int32),
        compiler_params=pltpu.CompilerParams(
            dimension_semantics=("arbitrary",)),
    )(pos, pos.T, batch_i32.reshape(n, 1), batch_i32.reshape(1, n))


def _sc_gather(table, idx_flat, chunk=128):
    e = idx_flat.shape[0]
    d = table.shape[1]
    nw = 32
    per_w = e // nw
    steps = per_w // chunk
    mesh = plsc.VectorSubcoreMesh(core_axis_name="c", subcore_axis_name="s")

    @functools.partial(
        pl.kernel,
        out_type=jax.ShapeDtypeStruct((e, d), jnp.float32),
        mesh=mesh,
        scratch_types=[
            pltpu.VMEM((chunk,), jnp.int32),
            pltpu.VMEM((chunk, d), jnp.float32),
            pltpu.SemaphoreType.DMA,
        ],
        compiler_params=pltpu.CompilerParams(use_tc_tiling_on_sc=False),
    )
    def gather_kernel(tbl_hbm, idx_hbm, out_hbm, idx_v, rows_v, sem):
        wid = lax.axis_index("s") * 2 + lax.axis_index("c")
        base = wid * per_w

        def body(j, carry):
            off = base + j * chunk
            pltpu.sync_copy(idx_hbm.at[pl.ds(off, chunk)], idx_v)
            pltpu.async_copy(tbl_hbm.at[idx_v], rows_v, sem).wait()
            pltpu.sync_copy(rows_v, out_hbm.at[pl.ds(off, chunk)])
            return carry

        lax.fori_loop(0, steps, body, 0)

    return gather_kernel(table, idx_flat)


def _ppf_t(pnjT, pniT):
    dx = pnjT[0:1, :] - pniT[0:1, :]
    dy = pnjT[1:2, :] - pniT[1:2, :]
    dz = pnjT[2:3, :] - pniT[2:3, :]
    nix, niy, niz = pniT[3:4, :], pniT[4:5, :], pniT[5:6, :]
    njx, njy, njz = pnjT[3:4, :], pnjT[4:5, :], pnjT[5:6, :]

    def safe_norm3(x, y, z):
        sq = x * x + y * y + z * z
        zero = sq < 1e-20
        return jnp.where(zero, 0.0, jnp.sqrt(jnp.where(zero, 1.0, sq)))

    def angle(ax, ay, az, bx, by, bz):
        cx = ay * bz - az * by
        cy = az * bx - ax * bz
        cz = ax * by - ay * bx
        cn = safe_norm3(cx, cy, cz)
        dot = ax * bx + ay * by + az * bz
        deg = (cn == 0.0) & (jnp.abs(dot) < 1e-12)
        return jnp.arctan2(cn, jnp.where(deg, 1.0, dot))

    f0 = safe_norm3(dx, dy, dz)
    f1 = angle(nix, niy, niz, dx, dy, dz)
    f2 = angle(njx, njy, njz, dx, dy, dz)
    f3 = angle(nix, niy, niz, njx, njy, njz)
    return jnp.concatenate([f0, f1, f2, f3], axis=0)


def _conv1_kernel(pnj_ref, pn_ref, w1_ref, b1_ref, w2_ref, b2_ref,
                  feats_ref, h_ref):
    blk = h_ref.shape[0]
    pnjT = jnp.transpose(pnj_ref[...])
    pnT = jnp.transpose(pn_ref[...])
    pniT = jnp.broadcast_to(pnT[:, :, None],
                            (16, blk, _K)).reshape(16, blk * _K)
    featsT = _ppf_t(pnjT, pniT)
    feats = jnp.transpose(featsT)
    b1 = b1_ref[...]
    b2 = b2_ref[...]
    w2 = w2_ref[...]
    m1 = jnp.dot(jax.nn.relu(
        jnp.dot(feats, w1_ref[...], preferred_element_type=jnp.float32)
        + b1), w2, preferred_element_type=jnp.float32) + b2
    mx = jnp.max(m1.reshape(blk, _K, 64), axis=1)
    m1_self = jnp.dot(jax.nn.relu(b1), w2,
                      preferred_element_type=jnp.float32) + b2
    feats_ref[...] = feats
    h_ref[...] = jax.nn.relu(jnp.maximum(mx, m1_self))


def _run_conv1(pnj, pn, w1, b1, w2, b2, blk):
    ne = pnj.shape[0]
    n = ne // _K
    grid = n // blk
    feats, h = pl.pallas_call(
        _conv1_kernel,
        grid=(grid,),
        in_specs=[
            pl.BlockSpec((blk * _K, 16), lambda i: (i, 0)),
            pl.BlockSpec((blk, 16), lambda i: (i, 0)),
            pl.BlockSpec((4, 64), lambda i: (0, 0)),
            pl.BlockSpec((1, 64), lambda i: (0, 0)),
            pl.BlockSpec((64, 64), lambda i: (0, 0)),
            pl.BlockSpec((1, 64), lambda i: (0, 0)),
        ],
        out_specs=[
            pl.BlockSpec((blk * _K, 4), lambda i: (i, 0)),
            pl.BlockSpec((blk, 64), lambda i: (i, 0)),
        ],
        out_shape=[
            jax.ShapeDtypeStruct((ne, 4), jnp.float32),
            jax.ShapeDtypeStruct((n, 64), jnp.float32),
        ],
        compiler_params=pltpu.CompilerParams(
            dimension_semantics=("arbitrary",)),
    )(pnj, pn, w1, b1, w2, b2)
    return feats, h


def _conv2_kernel(hg_ref, feats_ref, h_ref, batchT_ref,
                  w3h_ref, w3f_ref, b3_ref, w4_ref, b4_ref,
                  wc_ref, bc_ref,
                  s_ref, c_ref, out_ref):
    i = pl.program_id(0)
    nsteps = pl.num_programs(0)
    blk = h_ref.shape[0]

    w4 = w4_ref[...]
    b4 = b4_ref[...]
    a = jax.nn.relu(
        jnp.dot(hg_ref[...], w3h_ref[...], preferred_element_type=jnp.float32)
        + jnp.dot(feats_ref[...], w3f_ref[...],
                  preferred_element_type=jnp.float32)
        + b3_ref[...])
    m2 = jnp.dot(a, w4, preferred_element_type=jnp.float32) + b4
    mx = jnp.max(m2.reshape(blk, _K, 64), axis=1)

    a_self = jax.nn.relu(
        jnp.dot(h_ref[...], w3h_ref[...], preferred_element_type=jnp.float32)
        + b3_ref[...])
    m_self = jnp.dot(a_self, w4, preferred_element_type=jnp.float32) + b4
    h2 = jax.nn.relu(jnp.maximum(mx, m_self))

    ohT = (batchT_ref[...]
           == lax.broadcasted_iota(jnp.int32, (8, 1), 0)).astype(jnp.float32)
    part_s = jnp.dot(ohT, h2, preferred_element_type=jnp.float32)
    part_c = jnp.sum(ohT, axis=1, keepdims=True)

    @pl.when(i == 0)
    def _init():
        s_ref[...] = jnp.zeros_like(s_ref)
        c_ref[...] = jnp.zeros_like(c_ref)
        out_ref[...] = jnp.zeros_like(out_ref)

    s_ref[...] += part_s
    c_ref[...] += part_c

    @pl.when(i == nsteps - 1)
    def _final():
        pooled = s_ref[...] / jnp.maximum(c_ref[...], 1.0)
        out_ref[...] = jnp.dot(pooled, wc_ref[...],
                               preferred_element_type=jnp.float32) + bc_ref[...]


def _run_conv2(hg, feats, h, batch_i32, w3h, w3f, b3, w4, b4, wc, bc, blk):
    n = h.shape[0]
    classes = wc.shape[1]
    grid = n // blk
    _, _, out = pl.pallas_call(
        _conv2_kernel,
        grid=(grid,),
        in_specs=[
            pl.BlockSpec((blk * _K, 64), lambda i: (i, 0)),
            pl.BlockSpec((blk * _K, 4), lambda i: (i, 0)),
            pl.BlockSpec((blk, 64), lambda i: (i, 0)),
            pl.BlockSpec((1, blk), lambda i: (0, i)),
            pl.BlockSpec((64, 64), lambda i: (0, 0)),
            pl.BlockSpec((4, 64), lambda i: (0, 0)),
            pl.BlockSpec((1, 64), lambda i: (0, 0)),
            pl.BlockSpec((64, 64), lambda i: (0, 0)),
            pl.BlockSpec((1, 64), lambda i: (0, 0)),
            pl.BlockSpec((64, classes), lambda i: (0, 0)),
            pl.BlockSpec((1, classes), lambda i: (0, 0)),
        ],
        out_specs=[
            pl.BlockSpec((8, 64), lambda i: (0, 0)),
            pl.BlockSpec((8, 1), lambda i: (0, 0)),
            pl.BlockSpec((8, classes), lambda i: (0, 0)),
        ],
        out_shape=[
            jax.ShapeDtypeStruct((8, 64), jnp.float32),
            jax.ShapeDtypeStruct((8, 1), jnp.float32),
            jax.ShapeDtypeStruct((8, classes), jnp.float32),
        ],
        compiler_params=pltpu.CompilerParams(
            dimension_semantics=("arbitrary",)),
    )(hg, feats, h, batch_i32.reshape(1, n), w3h, w3f, b3, w4, b4, wc, bc)
    return out


def kernel(pos, batch, normal, W1, b1, W2, b2, W3, b3, W4, b4, Wc, bc):
    n = pos.shape[0]
    batch_i32 = batch.astype(jnp.int32)

    idx = _run_knn(pos, batch_i32, blk=1024)

    pn = jnp.concatenate(
        [pos, normal, jnp.zeros((n, 10), jnp.float32)], axis=1)
    src = idx.reshape(-1)
    pnj = _sc_gather(pn, src)

    feats, h = _run_conv1(pnj, pn, W1, b1.reshape(1, -1),
                          W2, b2.reshape(1, -1), blk=256)

    hg = _sc_gather(h, src)

    out = _run_conv2(
        hg, feats, h, batch_i32,
        W3[:64, :], W3[64:, :], b3.reshape(1, -1),
        W4, b4.reshape(1, -1), Wc, bc.reshape(1, -1),
        blk=256)
    return out


# speedup vs baseline: 1.1995x; 1.1995x over previous
import functools

import jax
import jax.numpy as jnp
from jax import lax
from jax.experimental import pallas as pl
from jax.experimental.pallas import tpu as pltpu
from jax.experimental.pallas import tpu_sc as plsc

_K = 16
_FMAX = 3.0e38
_CT = 512


def _knn_kernel(pos_ref, posT_ref, batch_ref, batchT_ref, idx_ref):
    i = pl.program_id(0)
    blk = pos_ref.shape[0]
    n = posT_ref.shape[1]

    pos_b = pos_ref[...]
    px = pos_b[:, 0:1]
    py = pos_b[:, 1:2]
    pz = pos_b[:, 2:3]
    sq_b = px * px + py * py + pz * pz
    batch_b = batch_ref[...]
    rowid = i * blk + lax.broadcasted_iota(jnp.int32, (blk, 1), 0)

    batch_row = batchT_ref[...]
    col_row = lax.broadcasted_iota(jnp.int32, (1, n), 1)
    bmin = jnp.min(batch_b)
    bmax = jnp.max(batch_b)
    start = jnp.min(jnp.where(batch_row == bmin, col_row, n))
    end = jnp.max(jnp.where(batch_row == bmax, col_row + 1, 0))
    t0 = start // _CT
    t1 = (end + _CT - 1) // _CT

    lane16 = lax.broadcasted_iota(jnp.int32, (blk, _K), 1)

    def body(j, carry):
        cur_val, cur_idx = carry
        c = j * _CT
        q = posT_ref[:, pl.ds(c, _CT)]
        qx = q[0:1, :]
        qy = q[1:2, :]
        qz = q[2:3, :]
        dot = px * qx + py * qy + pz * qz
        sq_t = qx * qx + qy * qy + qz * qz
        d = sq_b + sq_t - 2.0 * dot
        gidx = c + lax.broadcasted_iota(jnp.int32, (blk, _CT), 1)
        bt = batchT_ref[:, pl.ds(c, _CT)]
        invalid = (batch_b != bt) | (gidx == rowid)
        d = jnp.where(invalid, _FMAX, d)

        arr_val = jnp.concatenate([cur_val, d], axis=1)
        arr_idx = jnp.concatenate([cur_idx, gidx], axis=1)
        new_val = cur_val
        new_idx = cur_idx
        for r in range(_K):
            mv = jnp.min(arr_val, axis=1, keepdims=True)
            ci = jnp.min(jnp.where(arr_val == mv, arr_idx, jnp.int32(2**30)),
                         axis=1, keepdims=True)
            new_val = jnp.where(lane16 == r, mv, new_val)
            new_idx = jnp.where(lane16 == r, ci, new_idx)
            arr_val = jnp.where((arr_val == mv) & (arr_idx == ci),
                                float("inf"), arr_val)
        return new_val, new_idx

    cur_val = jnp.full((blk, _K), float("inf"), jnp.float32)
    cur_idx = jnp.full((blk, _K), 2**30, jnp.int32)
    _, cur_idx = lax.fori_loop(t0, t1, body, (cur_val, cur_idx))
    idx_ref[...] = cur_idx


def _run_knn(pos, batch_i32, blk):
    n = pos.shape[0]
    grid = n // blk
    return pl.pallas_call(
        _knn_kernel,
        grid=(grid,),
        in_specs=[
            pl.BlockSpec((blk, 3), lambda i: (i, 0)),
            pl.BlockSpec((3, n), lambda i: (0, 0)),
            pl.BlockSpec((blk, 1), lambda i: (i, 0)),
            pl.BlockSpec((1, n), lambda i: (0, 0)),
        ],
        out_specs=pl.BlockSpec((blk, _K), lambda i: (i, 0)),
        out_shape=jax.ShapeDtypeStruct((n, _K), jnp.int32),
        compiler_params=pltpu.CompilerParams(
            dimension_semantics=("arbitrary",)),
    )(pos, pos.T, batch_i32.reshape(n, 1), batch_i32.reshape(1, n))


def _sc_gather(table, idx_flat, chunk=128):
    e = idx_flat.shape[0]
    d = table.shape[1]
    nw = 32
    per_w = e // nw
    steps = per_w // chunk
    mesh = plsc.VectorSubcoreMesh(core_axis_name="c", subcore_axis_name="s")

    @functools.partial(
        pl.kernel,
        out_type=jax.ShapeDtypeStruct((e, d), jnp.float32),
        mesh=mesh,
        scratch_types=[
            pltpu.VMEM((per_w,), jnp.int32),
            pltpu.VMEM((chunk, d), jnp.float32),
            pltpu.VMEM((chunk, d), jnp.float32),
            pltpu.SemaphoreType.DMA,
            pltpu.SemaphoreType.DMA,
        ],
        compiler_params=pltpu.CompilerParams(use_tc_tiling_on_sc=False),
    )
    def gather_kernel(tbl_hbm, idx_hbm, out_hbm, idx_v, rows_a, rows_b,
                      sem_a, sem_b):
        wid = lax.axis_index("s") * 2 + lax.axis_index("c")
        base = wid * per_w
        pltpu.sync_copy(idx_hbm.at[pl.ds(base, per_w)], idx_v)

        def body(j2, carry):
            o0 = 2 * j2 * chunk
            o1 = o0 + chunk
            cp0 = pltpu.async_copy(
                tbl_hbm.at[idx_v.at[pl.ds(o0, chunk)]], rows_a, sem_a)
            cp1 = pltpu.async_copy(
                tbl_hbm.at[idx_v.at[pl.ds(o1, chunk)]], rows_b, sem_b)
            cp0.wait()
            pltpu.sync_copy(rows_a, out_hbm.at[pl.ds(base + o0, chunk)])
            cp1.wait()
            pltpu.sync_copy(rows_b, out_hbm.at[pl.ds(base + o1, chunk)])
            return carry

        lax.fori_loop(0, steps // 2, body, 0)

    return gather_kernel(table, idx_flat)


def _ppf_t(pnjT, pniT):
    dx = pnjT[0:1, :] - pniT[0:1, :]
    dy = pnjT[1:2, :] - pniT[1:2, :]
    dz = pnjT[2:3, :] - pniT[2:3, :]
    nix, niy, niz = pniT[3:4, :], pniT[4:5, :], pniT[5:6, :]
    njx, njy, njz = pnjT[3:4, :], pnjT[4:5, :], pnjT[5:6, :]

    def safe_norm3(x, y, z):
        sq = x * x + y * y + z * z
        zero = sq < 1e-20
        return jnp.where(zero, 0.0, jnp.sqrt(jnp.where(zero, 1.0, sq)))

    def angle(ax, ay, az, bx, by, bz):
        cx = ay * bz - az * by
        cy = az * bx - ax * bz
        cz = ax * by - ay * bx
        cn = safe_norm3(cx, cy, cz)
        dot = ax * bx + ay * by + az * bz
        deg = (cn == 0.0) & (jnp.abs(dot) < 1e-12)
        return jnp.arctan2(cn, jnp.where(deg, 1.0, dot))

    f0 = safe_norm3(dx, dy, dz)
    f1 = angle(nix, niy, niz, dx, dy, dz)
    f2 = angle(njx, njy, njz, dx, dy, dz)
    f3 = angle(nix, niy, niz, njx, njy, njz)
    return jnp.concatenate([f0, f1, f2, f3], axis=0)


def _conv1_kernel(pnj_ref, pn_ref, w1_ref, b1_ref, w2_ref, b2_ref,
                  feats_ref, h_ref):
    blk = h_ref.shape[0]
    pnjT = jnp.transpose(pnj_ref[...])
    pnT = jnp.transpose(pn_ref[...])
    pniT = jnp.broadcast_to(pnT[:, :, None],
                            (16, blk, _K)).reshape(16, blk * _K)
    featsT = _ppf_t(pnjT, pniT)
    feats = jnp.transpose(featsT)
    b1 = b1_ref[...]
    b2 = b2_ref[...]
    w2 = w2_ref[...]
    m1 = jnp.dot(jax.nn.relu(
        jnp.dot(feats, w1_ref[...], preferred_element_type=jnp.float32)
        + b1), w2, preferred_element_type=jnp.float32) + b2
    mx = jnp.max(m1.reshape(blk, _K, 64), axis=1)
    m1_self = jnp.dot(jax.nn.relu(b1), w2,
                      preferred_element_type=jnp.float32) + b2
    feats_ref[...] = feats
    h_ref[...] = jax.nn.relu(jnp.maximum(mx, m1_self))


def _run_conv1(pnj, pn, w1, b1, w2, b2, blk):
    ne = pnj.shape[0]
    n = ne // _K
    grid = n // blk
    feats, h = pl.pallas_call(
        _conv1_kernel,
        grid=(grid,),
        in_specs=[
            pl.BlockSpec((blk * _K, 16), lambda i: (i, 0)),
            pl.BlockSpec((blk, 16), lambda i: (i, 0)),
            pl.BlockSpec((4, 64), lambda i: (0, 0)),
            pl.BlockSpec((1, 64), lambda i: (0, 0)),
            pl.BlockSpec((64, 64), lambda i: (0, 0)),
            pl.BlockSpec((1, 64), lambda i: (0, 0)),
        ],
        out_specs=[
            pl.BlockSpec((blk * _K, 4), lambda i: (i, 0)),
            pl.BlockSpec((blk, 64), lambda i: (i, 0)),
        ],
        out_shape=[
            jax.ShapeDtypeStruct((ne, 4), jnp.float32),
            jax.ShapeDtypeStruct((n, 64), jnp.float32),
        ],
        compiler_params=pltpu.CompilerParams(
            dimension_semantics=("arbitrary",)),
    )(pnj, pn, w1, b1, w2, b2)
    return feats, h


def _conv2_kernel(hg_ref, feats_ref, h_ref, batchT_ref,
                  w3h_ref, w3f_ref, b3_ref, w4_ref, b4_ref,
                  wc_ref, bc_ref,
                  s_ref, c_ref, out_ref):
    i = pl.program_id(0)
    nsteps = pl.num_programs(0)
    blk = h_ref.shape[0]

    w4 = w4_ref[...]
    b4 = b4_ref[...]
    a = jax.nn.relu(
        jnp.dot(hg_ref[...], w3h_ref[...], preferred_element_type=jnp.float32)
        + jnp.dot(feats_ref[...], w3f_ref[...],
                  preferred_element_type=jnp.float32)
        + b3_ref[...])
    m2 = jnp.dot(a, w4, preferred_element_type=jnp.float32) + b4
    mx = jnp.max(m2.reshape(blk, _K, 64), axis=1)

    a_self = jax.nn.relu(
        jnp.dot(h_ref[...], w3h_ref[...], preferred_element_type=jnp.float32)
        + b3_ref[...])
    m_self = jnp.dot(a_self, w4, preferred_element_type=jnp.float32) + b4
    h2 = jax.nn.relu(jnp.maximum(mx, m_self))

    ohT = (batchT_ref[...]
           == lax.broadcasted_iota(jnp.int32, (8, 1), 0)).astype(jnp.float32)
    part_s = jnp.dot(ohT, h2, preferred_element_type=jnp.float32)
    part_c = jnp.sum(ohT, axis=1, keepdims=True)

    @pl.when(i == 0)
    def _init():
        s_ref[...] = jnp.zeros_like(s_ref)
        c_ref[...] = jnp.zeros_like(c_ref)
        out_ref[...] = jnp.zeros_like(out_ref)

    s_ref[...] += part_s
    c_ref[...] += part_c

    @pl.when(i == nsteps - 1)
    def _final():
        pooled = s_ref[...] / jnp.maximum(c_ref[...], 1.0)
        out_ref[...] = jnp.dot(pooled, wc_ref[...],
                               preferred_element_type=jnp.float32) + bc_ref[...]


def _run_conv2(hg, feats, h, batch_i32, w3h, w3f, b3, w4, b4, wc, bc, blk):
    n = h.shape[0]
    classes = wc.shape[1]
    grid = n // blk
    _, _, out = pl.pallas_call(
        _conv2_kernel,
        grid=(grid,),
        in_specs=[
            pl.BlockSpec((blk * _K, 64), lambda i: (i, 0)),
            pl.BlockSpec((blk * _K, 4), lambda i: (i, 0)),
            pl.BlockSpec((blk, 64), lambda i: (i, 0)),
            pl.BlockSpec((1, blk), lambda i: (0, i)),
            pl.BlockSpec((64, 64), lambda i: (0, 0)),
            pl.BlockSpec((4, 64), lambda i: (0, 0)),
            pl.BlockSpec((1, 64), lambda i: (0, 0)),
            pl.BlockSpec((64, 64), lambda i: (0, 0)),
            pl.BlockSpec((1, 64), lambda i: (0, 0)),
            pl.BlockSpec((64, classes), lambda i: (0, 0)),
            pl.BlockSpec((1, classes), lambda i: (0, 0)),
        ],
        out_specs=[
            pl.BlockSpec((8, 64), lambda i: (0, 0)),
            pl.BlockSpec((8, 1), lambda i: (0, 0)),
            pl.BlockSpec((8, classes), lambda i: (0, 0)),
        ],
        out_shape=[
            jax.ShapeDtypeStruct((8, 64), jnp.float32),
            jax.ShapeDtypeStruct((8, 1), jnp.float32),
            jax.ShapeDtypeStruct((8, classes), jnp.float32),
        ],
        compiler_params=pltpu.CompilerParams(
            dimension_semantics=("arbitrary",)),
    )(hg, feats, h, batch_i32.reshape(1, n), w3h, w3f, b3, w4, b4, wc, bc)
    return out


def kernel(pos, batch, normal, W1, b1, W2, b2, W3, b3, W4, b4, Wc, bc):
    n = pos.shape[0]
    batch_i32 = batch.astype(jnp.int32)

    idx = _run_knn(pos, batch_i32, blk=512)

    pn = jnp.concatenate(
        [pos, normal, jnp.zeros((n, 10), jnp.float32)], axis=1)
    src = idx.reshape(-1)
    pnj = _sc_gather(pn, src)

    feats, h = _run_conv1(pnj, pn, W1, b1.reshape(1, -1),
                          W2, b2.reshape(1, -1), blk=256)

    hg = _sc_gather(h, src)

    out = _run_conv2(
        hg, feats, h, batch_i32,
        W3[:64, :], W3[64:, :], b3.reshape(1, -1),
        W4, b4.reshape(1, -1), Wc, bc.reshape(1, -1),
        blk=256)
    return out
